# baseline (speedup 1.0000x reference)
import jax
import jax.numpy as jnp
from jax.experimental import pallas as pl

N = 10000
NGRAPHS = 64


def _mlp_body(g_ref, w0, b0, w1, b1, w2, b2, w3, b3, out_ref):
    g = g_ref[...]
    g = jnp.maximum(jnp.dot(g, w0[...]) + b0[...], 0.0)
    g = jnp.maximum(jnp.dot(g, w1[...]) + b1[...], 0.0)
    g = jnp.maximum(jnp.dot(g, w2[...]) + b2[...], 0.0)
    out_ref[...] = jnp.dot(g, w3[...]) + b3[...]


def kernel(x, edge_index, edge_attr, batch, w_rel0, b_rel0, w_root0, w_rel1, b_rel1, w_root1, w_rel2, b_rel2, w_root2, w_rel3, b_rel3, w_root3, w_rel4, b_rel4, w_root4, w_mlp0, b_mlp0, w_mlp1, b_mlp1, w_mlp2, b_mlp2, w_mlp3, b_mlp3):
    conv_params = [(w_rel0, b_rel0, w_root0), (w_rel1, b_rel1, w_root1), (w_rel2, b_rel2, w_root2), (w_rel3, b_rel3, w_root3), (w_rel4, b_rel4, w_root4)]
    src = edge_index[0]
    dst = edge_index[1]
    h = x
    for (w_rel, b_rel, w_root) in conv_params:
        msg = edge_attr[:, None] * jnp.take(h, src, axis=0)
        agg = jax.ops.segment_sum(msg, dst, num_segments=N)
        h = agg @ w_rel.T + b_rel + h @ w_root.T
        h = jax.nn.relu(h)
    sums = jax.ops.segment_sum(h, batch, num_segments=NGRAPHS)
    counts = jax.ops.segment_sum(jnp.ones((N,), dtype=h.dtype), batch, num_segments=NGRAPHS)
    g = sums / jnp.maximum(counts, 1.0)[:, None]
    out = pl.pallas_call(
        _mlp_body,
        out_shape=jax.ShapeDtypeStruct((NGRAPHS, 1), jnp.float32),
    )(g, w_mlp0.T, b_mlp0.reshape(1, -1), w_mlp1.T, b_mlp1.reshape(1, -1),
      w_mlp2.T, b_mlp2.reshape(1, -1), w_mlp3.T, b_mlp3.reshape(1, -1))
    return out


# speedup vs baseline: 5.7403x; 5.7403x over previous
import functools

import jax
import jax.numpy as jnp
from jax import lax
from jax.experimental import pallas as pl
from jax.experimental.pallas import tpu as pltpu
from jax.experimental.pallas import tpu_sc as plsc

N = 10000
E = 320000
NGRAPHS = 64

NC = 2
NS = 16
NW = NC * NS
EPAD = 327680
EPW = EPAD // NW
B = 1024
NCHUNK = EPW // B
NPAD = 10240
SUBN = NPAD // NS


_GDN = lax.GatherDimensionNumbers(offset_dims=(), collapsed_slice_dims=(0,),
                                  start_index_map=(0,))


def _bcast_lane(v, lane):
    idx = jnp.full((16, 1), lane, jnp.int32)
    return lax.gather(v, idx, _GDN, (1,),
                      mode=lax.GatherScatterMode.PROMISE_IN_BOUNDS)


def _edge_agg_body(d, hw_hbm, src_hbm, dst_hbm, attr_hbm, out_hbm,
                   acc_sh, src_v, dst_v, attr_v, rows_v, sem):
    cid = lax.axis_index("c")
    sid = lax.axis_index("s")
    wid = cid * NS + sid

    def zbody(i, c):
        for j in range(d // 16):
            rows_v[i, pl.ds(j * 16, 16)] = jnp.zeros((16,), jnp.float32)
        return c
    lax.fori_loop(0, SUBN, zbody, 0)
    pltpu.sync_copy(rows_v.at[pl.ds(0, SUBN)], acc_sh.at[pl.ds(sid * SUBN, SUBN)])
    plsc.subcore_barrier()

    @pl.loop(0, NCHUNK)
    def chunk_loop(c):
        base = wid * EPW + c * B
        pltpu.sync_copy(src_hbm.at[pl.ds(base, B)], src_v)
        pltpu.sync_copy(attr_hbm.at[pl.ds(base, B)], attr_v)
        dbase = wid * (EPW // 128) + c * (B // 128)
        pltpu.sync_copy(dst_hbm.at[pl.ds(dbase, B // 128)], dst_v)
        pltpu.async_copy(hw_hbm.at[src_v], rows_v, sem).wait()

        def mbody(g, c2):
            e0 = g * 16
            a16 = attr_v[pl.ds(e0, 16)]
            for e16 in range(16):
                a = _bcast_lane(a16, e16)
                for j in range(d // 16):
                    sl = pl.ds(j * 16, 16)
                    rows_v[e0 + e16, sl] = rows_v[e0 + e16, sl] * a
            return c2
        lax.fori_loop(0, B // 16, mbody, 0)

        for j in range(B // 128):
            pltpu.sync_copy(rows_v.at[pl.ds(j * 128, 128)],
                            acc_sh.at[dst_v.at[j]], add=True)

    plsc.subcore_barrier()
    pltpu.sync_copy(acc_sh.at[pl.ds(sid * SUBN, SUBN)],
                    out_hbm.at[pl.ds(cid * NPAD + sid * SUBN, SUBN)])


@functools.cache
def _make_edge_agg(d):
    mesh = plsc.VectorSubcoreMesh(core_axis_name="c", subcore_axis_name="s",
                                  num_cores=NC, num_subcores=NS)
    return pl.kernel(
        functools.partial(_edge_agg_body, d),
        out_type=jax.ShapeDtypeStruct((NC * NPAD, d), jnp.float32),
        mesh=mesh,
        compiler_params=pltpu.CompilerParams(use_tc_tiling_on_sc=False),
        scratch_types=[
            pltpu.VMEM_SHARED((NPAD, d), jnp.float32),
            pltpu.VMEM((B,), jnp.int32),
            pltpu.VMEM((B // 128, 128), jnp.int32),
            pltpu.VMEM((B,), jnp.float32),
            pltpu.VMEM((B, d), jnp.float32),
            pltpu.SemaphoreType.DMA,
        ],
    )


def _dot(a, b):
    return lax.dot_general(a, b, (((1,), (0,)), ((), ())),
                           preferred_element_type=jnp.float32)


def _mm_body(x_ref, w_ref, o_ref):
    o_ref[...] = _dot(x_ref[...], w_ref[...])


def _combine_pre_body(p_ref, h_ref, wroot_ref, b_ref, o_ref):
    agg = p_ref[pl.ds(0, N), :] + p_ref[pl.ds(NPAD, N), :]
    o_ref[...] = jnp.maximum(agg + b_ref[...] + _dot(h_ref[...], wroot_ref[...]), 0.0)


def _combine_post_body(p_ref, h_ref, wrel_ref, wroot_ref, b_ref, o_ref):
    agg = p_ref[pl.ds(0, N), :] + p_ref[pl.ds(NPAD, N), :]
    o_ref[...] = jnp.maximum(
        _dot(agg, wrel_ref[...]) + b_ref[...] + _dot(h_ref[...], wroot_ref[...]), 0.0)


def _combine_pre_mm_body(p_ref, h_ref, wroot_ref, b_ref, wnext_ref, o_ref, hw_ref):
    agg = p_ref[pl.ds(0, N), :] + p_ref[pl.ds(NPAD, N), :]
    h = jnp.maximum(agg + b_ref[...] + _dot(h_ref[...], wroot_ref[...]), 0.0)
    o_ref[...] = h
    hw_ref[...] = _dot(h, wnext_ref[...])


def _combine_post_mm_body(p_ref, h_ref, wrel_ref, wroot_ref, b_ref, wnext_ref,
                          o_ref, hw_ref):
    agg = p_ref[pl.ds(0, N), :] + p_ref[pl.ds(NPAD, N), :]
    h = jnp.maximum(
        _dot(agg, wrel_ref[...]) + b_ref[...] + _dot(h_ref[...], wroot_ref[...]), 0.0)
    o_ref[...] = h
    hw_ref[...] = _dot(h, wnext_ref[...])


def _head_body(p_ref, h_ref, wroot_ref, b_ref, batch_ref,
               w0, b0, w1, b1, w2, b2, w3, b3, o_ref):
    agg = p_ref[pl.ds(0, N), :] + p_ref[pl.ds(NPAD, N), :]
    h = jnp.maximum(agg + b_ref[...] + _dot(h_ref[...], wroot_ref[...]), 0.0)
    gid = lax.broadcasted_iota(jnp.int32, (NGRAPHS, N), 0)
    onehot = (gid == batch_ref[...]).astype(jnp.float32)
    sums = _dot(onehot, h)
    counts = jnp.sum(onehot, axis=1, keepdims=True)
    g = sums / jnp.maximum(counts, 1.0)
    g = jnp.maximum(_dot(g, w0[...]) + b0[...], 0.0)
    g = jnp.maximum(_dot(g, w1[...]) + b1[...], 0.0)
    g = jnp.maximum(_dot(g, w2[...]) + b2[...], 0.0)
    o_ref[...] = _dot(g, w3[...]) + b3[...]


def _tc(body, out_shapes):
    return pl.pallas_call(body, out_shape=out_shapes)


def kernel(x, edge_index, edge_attr, batch, w_rel0, b_rel0, w_root0, w_rel1, b_rel1, w_root1, w_rel2, b_rel2, w_root2, w_rel3, b_rel3, w_root3, w_rel4, b_rel4, w_root4, w_mlp0, b_mlp0, w_mlp1, b_mlp1, w_mlp2, b_mlp2, w_mlp3, b_mlp3):
    f32 = jnp.float32
    src = edge_index[0]
    dst = edge_index[1]
    pad = EPAD - E
    srcp = jnp.concatenate([src, jnp.zeros((pad,), jnp.int32)])
    attrp = jnp.concatenate([edge_attr, jnp.zeros((pad,), f32)])
    dstp = jnp.concatenate([dst, jnp.zeros((pad,), jnp.int32)]).reshape(EPAD // 128, 128)
    batch2d = batch.reshape(1, N)

    shp = lambda m, n: jax.ShapeDtypeStruct((m, n), f32)

    hw0 = _tc(_mm_body, shp(N, 32))(x, w_rel0.T)
    p0 = _make_edge_agg(32)(hw0, srcp, dstp, attrp)
    h1 = _tc(_combine_pre_body, shp(N, 32))(p0, x, w_root0.T, b_rel0.reshape(1, -1))

    p1 = _make_edge_agg(32)(h1, srcp, dstp, attrp)
    h2 = _tc(_combine_post_body, shp(N, 64))(
        p1, h1, w_rel1.T, w_root1.T, b_rel1.reshape(1, -1))

    p2 = _make_edge_agg(64)(h2, srcp, dstp, attrp)
    h3, hw3 = _tc(_combine_post_mm_body, (shp(N, 128), shp(N, 64)))(
        p2, h2, w_rel2.T, w_root2.T, b_rel2.reshape(1, -1), w_rel3.T)

    p3 = _make_edge_agg(64)(hw3, srcp, dstp, attrp)
    h4, hw4 = _tc(_combine_pre_mm_body, (shp(N, 64), shp(N, 32)))(
        p3, h3, w_root3.T, b_rel3.reshape(1, -1), w_rel4.T)

    p4 = _make_edge_agg(32)(hw4, srcp, dstp, attrp)

    out = _tc(_head_body, shp(NGRAPHS, 1))(
        p4, h4, w_root4.T, b_rel4.reshape(1, -1), batch2d,
        w_mlp0.T, b_mlp0.reshape(1, -1), w_mlp1.T, b_mlp1.reshape(1, -1),
        w_mlp2.T, b_mlp2.reshape(1, -1), w_mlp3.T, b_mlp3.reshape(1, -1))
    return out


# speedup vs baseline: 6.5662x; 1.1439x over previous
import functools

import jax
import jax.numpy as jnp
from jax import lax
from jax.experimental import pallas as pl
from jax.experimental.pallas import tpu as pltpu
from jax.experimental.pallas import tpu_sc as plsc

N = 10000
E = 320000
NGRAPHS = 64

NC = 2
NS = 16
NW = NC * NS
EPAD = 327680
EPW = EPAD // NW
B = 1024
NCHUNK = EPW // B
NPAD = 10240
SUBN = NPAD // NS


_GDN = lax.GatherDimensionNumbers(offset_dims=(), collapsed_slice_dims=(0,),
                                  start_index_map=(0,))


def _bcast_lane(v, lane):
    idx = jnp.full((16, 1), lane, jnp.int32)
    return lax.gather(v, idx, _GDN, (1,),
                      mode=lax.GatherScatterMode.PROMISE_IN_BOUNDS)


def _edge_agg_body(d, B, hw_hbm, src_hbm, dst_hbm, attr_hbm, out_hbm,
                   acc_sh, src_v, dst_v, attr_v, rows0, rows1,
                   gs0, gs1, ss0, ss1):
    NCH = EPW // B
    cid = lax.axis_index("c")
    sid = lax.axis_index("s")
    wid = cid * NS + sid
    rows = (rows0, rows1)
    gsem = (gs0, gs1)
    ssem = (ss0, ss1)

    pltpu.sync_copy(src_hbm.at[pl.ds(wid * EPW, EPW)], src_v)
    pltpu.sync_copy(attr_hbm.at[pl.ds(wid * EPW, EPW)], attr_v)
    pltpu.sync_copy(dst_hbm.at[pl.ds(wid * (EPW // 128), EPW // 128)], dst_v)

    nz = min(SUBN, B)

    def zbody(i, c):
        for j in range(d // 16):
            rows0[i, pl.ds(j * 16, 16)] = jnp.zeros((16,), jnp.float32)
        return c
    lax.fori_loop(0, nz, zbody, 0)
    off = 0
    while off < SUBN:
        cur = min(nz, SUBN - off)
        pltpu.sync_copy(rows0.at[pl.ds(0, cur)],
                        acc_sh.at[pl.ds(sid * SUBN + off, cur)])
        off += cur
    plsc.subcore_barrier()

    def fire_gather(c, k):
        pltpu.async_copy(hw_hbm.at[src_v.at[pl.ds(c * B, B)]], rows[k], gsem[k])

    def wait_gather(k):
        pltpu.make_async_copy(hw_hbm.at[src_v.at[pl.ds(0, B)]],
                              rows[k], gsem[k]).wait()

    def multiply(c, k):
        rk = rows[k]

        def mbody(g, carry):
            a16 = attr_v[pl.ds(c * B + g * 16, 16)]
            for e16 in range(16):
                a = _bcast_lane(a16, e16)
                e = g * 16 + e16
                for j in range(d // 16):
                    sl = pl.ds(j * 16, 16)
                    rk[e, sl] = rk[e, sl] * a
            return carry
        lax.fori_loop(0, B // 16, mbody, 0)

    def fire_scatters(c, k):
        for j in range(B // 128):
            pltpu.async_copy(rows[k].at[pl.ds(j * 128, 128)],
                             acc_sh.at[dst_v.at[c * (B // 128) + j]],
                             ssem[k], add=True)

    def drain_scatters(k):
        for j in range(B // 128):
            pltpu.make_async_copy(rows[k].at[pl.ds(j * 128, 128)],
                                  acc_sh.at[dst_v.at[0]], ssem[k]).wait()

    fire_gather(0, 0)
    fire_gather(1, 1)

    @pl.loop(0, NCH // 2)
    def round_loop(r):
        c0 = r * 2
        for k in (0, 1):
            wait_gather(k)
            multiply(c0 + k, k)
            fire_scatters(c0 + k, k)
        for k in (0, 1):
            drain_scatters(k)
            cn = jnp.minimum(c0 + k + 2, NCH - 1)
            fire_gather(cn, k)

    wait_gather(0)
    wait_gather(1)
    plsc.subcore_barrier()
    pltpu.sync_copy(acc_sh.at[pl.ds(sid * SUBN, SUBN)],
                    out_hbm.at[pl.ds(cid * NPAD + sid * SUBN, SUBN)])


@functools.cache
def _make_edge_agg(d):
    B = 1024 if d <= 32 else 256
    mesh = plsc.VectorSubcoreMesh(core_axis_name="c", subcore_axis_name="s",
                                  num_cores=NC, num_subcores=NS)
    return pl.kernel(
        functools.partial(_edge_agg_body, d, B),
        out_type=jax.ShapeDtypeStruct((NC * NPAD, d), jnp.float32),
        mesh=mesh,
        compiler_params=pltpu.CompilerParams(use_tc_tiling_on_sc=False),
        scratch_types=[
            pltpu.VMEM_SHARED((NPAD, d), jnp.float32),
            pltpu.VMEM((EPW,), jnp.int32),
            pltpu.VMEM((EPW // 128, 128), jnp.int32),
            pltpu.VMEM((EPW,), jnp.float32),
            pltpu.VMEM((B, d), jnp.float32),
            pltpu.VMEM((B, d), jnp.float32),
            pltpu.SemaphoreType.DMA,
            pltpu.SemaphoreType.DMA,
            pltpu.SemaphoreType.DMA,
            pltpu.SemaphoreType.DMA,
        ],
    )


def _dot(a, b):
    return lax.dot_general(a, b, (((1,), (0,)), ((), ())),
                           preferred_element_type=jnp.float32)


def _mm_body(x_ref, w_ref, o_ref):
    o_ref[...] = _dot(x_ref[...], w_ref[...])


def _combine_pre_body(p_ref, h_ref, wroot_ref, b_ref, o_ref):
    agg = p_ref[pl.ds(0, N), :] + p_ref[pl.ds(NPAD, N), :]
    o_ref[...] = jnp.maximum(agg + b_ref[...] + _dot(h_ref[...], wroot_ref[...]), 0.0)


def _combine_post_body(p_ref, h_ref, wrel_ref, wroot_ref, b_ref, o_ref):
    agg = p_ref[pl.ds(0, N), :] + p_ref[pl.ds(NPAD, N), :]
    o_ref[...] = jnp.maximum(
        _dot(agg, wrel_ref[...]) + b_ref[...] + _dot(h_ref[...], wroot_ref[...]), 0.0)


def _combine_pre_mm_body(p_ref, h_ref, wroot_ref, b_ref, wnext_ref, o_ref, hw_ref):
    agg = p_ref[pl.ds(0, N), :] + p_ref[pl.ds(NPAD, N), :]
    h = jnp.maximum(agg + b_ref[...] + _dot(h_ref[...], wroot_ref[...]), 0.0)
    o_ref[...] = h
    hw_ref[...] = _dot(h, wnext_ref[...])


def _combine_post_mm_body(p_ref, h_ref, wrel_ref, wroot_ref, b_ref, wnext_ref,
                          o_ref, hw_ref):
    agg = p_ref[pl.ds(0, N), :] + p_ref[pl.ds(NPAD, N), :]
    h = jnp.maximum(
        _dot(agg, wrel_ref[...]) + b_ref[...] + _dot(h_ref[...], wroot_ref[...]), 0.0)
    o_ref[...] = h
    hw_ref[...] = _dot(h, wnext_ref[...])


def _head_body(p_ref, h_ref, wroot_ref, b_ref, batch_ref,
               w0, b0, w1, b1, w2, b2, w3, b3, o_ref):
    agg = p_ref[pl.ds(0, N), :] + p_ref[pl.ds(NPAD, N), :]
    h = jnp.maximum(agg + b_ref[...] + _dot(h_ref[...], wroot_ref[...]), 0.0)
    gid = lax.broadcasted_iota(jnp.int32, (NGRAPHS, N), 0)
    onehot = (gid == batch_ref[...]).astype(jnp.float32)
    sums = _dot(onehot, h)
    counts = jnp.sum(onehot, axis=1, keepdims=True)
    g = sums / jnp.maximum(counts, 1.0)
    g = jnp.maximum(_dot(g, w0[...]) + b0[...], 0.0)
    g = jnp.maximum(_dot(g, w1[...]) + b1[...], 0.0)
    g = jnp.maximum(_dot(g, w2[...]) + b2[...], 0.0)
    o_ref[...] = _dot(g, w3[...]) + b3[...]


def _tc(body, out_shapes):
    return pl.pallas_call(body, out_shape=out_shapes)


def kernel(x, edge_index, edge_attr, batch, w_rel0, b_rel0, w_root0, w_rel1, b_rel1, w_root1, w_rel2, b_rel2, w_root2, w_rel3, b_rel3, w_root3, w_rel4, b_rel4, w_root4, w_mlp0, b_mlp0, w_mlp1, b_mlp1, w_mlp2, b_mlp2, w_mlp3, b_mlp3):
    f32 = jnp.float32
    src = edge_index[0]
    dst = edge_index[1]
    pad = EPAD - E
    srcp = jnp.concatenate([src, jnp.zeros((pad,), jnp.int32)])
    attrp = jnp.concatenate([edge_attr, jnp.zeros((pad,), f32)])
    dstp = jnp.concatenate([dst, jnp.zeros((pad,), jnp.int32)]).reshape(EPAD // 128, 128)
    batch2d = batch.reshape(1, N)

    shp = lambda m, n: jax.ShapeDtypeStruct((m, n), f32)

    hw0 = _tc(_mm_body, shp(N, 32))(x, w_rel0.T)
    p0 = _make_edge_agg(32)(hw0, srcp, dstp, attrp)
    h1 = _tc(_combine_pre_body, shp(N, 32))(p0, x, w_root0.T, b_rel0.reshape(1, -1))

    p1 = _make_edge_agg(32)(h1, srcp, dstp, attrp)
    h2 = _tc(_combine_post_body, shp(N, 64))(
        p1, h1, w_rel1.T, w_root1.T, b_rel1.reshape(1, -1))

    p2 = _make_edge_agg(64)(h2, srcp, dstp, attrp)
    h3, hw3 = _tc(_combine_post_mm_body, (shp(N, 128), shp(N, 64)))(
        p2, h2, w_rel2.T, w_root2.T, b_rel2.reshape(1, -1), w_rel3.T)

    p3 = _make_edge_agg(64)(hw3, srcp, dstp, attrp)
    h4, hw4 = _tc(_combine_pre_mm_body, (shp(N, 64), shp(N, 32)))(
        p3, h3, w_root3.T, b_rel3.reshape(1, -1), w_rel4.T)

    p4 = _make_edge_agg(32)(hw4, srcp, dstp, attrp)

    out = _tc(_head_body, shp(NGRAPHS, 1))(
        p4, h4, w_root4.T, b_rel4.reshape(1, -1), batch2d,
        w_mlp0.T, b_mlp0.reshape(1, -1), w_mlp1.T, b_mlp1.reshape(1, -1),
        w_mlp2.T, b_mlp2.reshape(1, -1), w_mlp3.T, b_mlp3.reshape(1, -1))
    return out


# speedup vs baseline: 7.4545x; 1.1353x over previous
import functools

import jax
import jax.numpy as jnp
from jax import lax
from jax.experimental import pallas as pl
from jax.experimental.pallas import tpu as pltpu
from jax.experimental.pallas import tpu_sc as plsc

N = 10000
E = 320000
NGRAPHS = 64

NC = 2
NS = 16
NW = NC * NS
EPAD = 327680
EPW = EPAD // NW
B = 1024
NCHUNK = EPW // B
NPAD = 10240
SUBN = NPAD // NS


_GDN = lax.GatherDimensionNumbers(offset_dims=(), collapsed_slice_dims=(0,),
                                  start_index_map=(0,))


def _bcast_lane(v, lane):
    idx = jnp.full((16, 1), lane, jnp.int32)
    return lax.gather(v, idx, _GDN, (1,),
                      mode=lax.GatherScatterMode.PROMISE_IN_BOUNDS)


def _edge_agg_body(d, B, hw_hbm, src_hbm, dst_hbm, attr_hbm, out_hbm,
                   acc_sh, src_v, dst_v, attr_v, rows0, rows1,
                   gs0, gs1, ss0, ss1):
    NCH = EPW // B
    cid = lax.axis_index("c")
    sid = lax.axis_index("s")
    wid = cid * NS + sid
    rows = (rows0, rows1)
    gsem = (gs0, gs1)
    ssem = (ss0, ss1)

    pltpu.sync_copy(src_hbm.at[pl.ds(wid * EPW, EPW)], src_v)
    pltpu.sync_copy(attr_hbm.at[pl.ds(wid * EPW, EPW)], attr_v)
    pltpu.sync_copy(dst_hbm.at[pl.ds(wid * (EPW // 128), EPW // 128)], dst_v)

    nz = min(SUBN, B)

    @plsc.parallel_loop(0, nz)
    def zbody(i):
        for j in range(d // 16):
            rows0[i, pl.ds(j * 16, 16)] = jnp.zeros((16,), jnp.float32)
    off = 0
    while off < SUBN:
        cur = min(nz, SUBN - off)
        pltpu.sync_copy(rows0.at[pl.ds(0, cur)],
                        acc_sh.at[pl.ds(sid * SUBN + off, cur)])
        off += cur
    plsc.subcore_barrier()

    def fire_gather(c, k):
        pltpu.async_copy(hw_hbm.at[src_v.at[pl.ds(c * B, B)]], rows[k], gsem[k])

    def wait_gather(k):
        pltpu.make_async_copy(hw_hbm.at[src_v.at[pl.ds(0, B)]],
                              rows[k], gsem[k]).wait()

    def multiply(c, k):
        rk = rows[k]

        @plsc.parallel_loop(0, B // 16, unroll=2)
        def mbody(g):
            a16 = attr_v[pl.ds(c * B + g * 16, 16)]
            for e16 in range(16):
                a = _bcast_lane(a16, e16)
                e = g * 16 + e16
                for j in range(d // 16):
                    sl = pl.ds(j * 16, 16)
                    rk[e, sl] = rk[e, sl] * a

    def fire_scatters(c, k):
        for j in range(B // 128):
            pltpu.async_copy(rows[k].at[pl.ds(j * 128, 128)],
                             acc_sh.at[dst_v.at[c * (B // 128) + j]],
                             ssem[k], add=True)

    def drain_scatters(k):
        for j in range(B // 128):
            pltpu.make_async_copy(rows[k].at[pl.ds(j * 128, 128)],
                                  acc_sh.at[dst_v.at[0]], ssem[k]).wait()

    fire_gather(0, 0)
    fire_gather(1, 1)

    @pl.loop(0, NCH // 2)
    def round_loop(r):
        c0 = r * 2
        for k in (0, 1):
            wait_gather(k)
            multiply(c0 + k, k)
            fire_scatters(c0 + k, k)
        for k in (0, 1):
            drain_scatters(k)
            cn = jnp.minimum(c0 + k + 2, NCH - 1)
            fire_gather(cn, k)

    wait_gather(0)
    wait_gather(1)
    plsc.subcore_barrier()
    pltpu.sync_copy(acc_sh.at[pl.ds(sid * SUBN, SUBN)],
                    out_hbm.at[pl.ds(cid * NPAD + sid * SUBN, SUBN)])


@functools.cache
def _make_edge_agg(d):
    B = 1024 if d <= 32 else 256
    mesh = plsc.VectorSubcoreMesh(core_axis_name="c", subcore_axis_name="s",
                                  num_cores=NC, num_subcores=NS)
    return pl.kernel(
        functools.partial(_edge_agg_body, d, B),
        out_type=jax.ShapeDtypeStruct((NC * NPAD, d), jnp.float32),
        mesh=mesh,
        compiler_params=pltpu.CompilerParams(use_tc_tiling_on_sc=False),
        scratch_types=[
            pltpu.VMEM_SHARED((NPAD, d), jnp.float32),
            pltpu.VMEM((EPW,), jnp.int32),
            pltpu.VMEM((EPW // 128, 128), jnp.int32),
            pltpu.VMEM((EPW,), jnp.float32),
            pltpu.VMEM((B, d), jnp.float32),
            pltpu.VMEM((B, d), jnp.float32),
            pltpu.SemaphoreType.DMA,
            pltpu.SemaphoreType.DMA,
            pltpu.SemaphoreType.DMA,
            pltpu.SemaphoreType.DMA,
        ],
    )


def _dot(a, b):
    return lax.dot_general(a, b, (((1,), (0,)), ((), ())),
                           preferred_element_type=jnp.float32)


def _mm_body(x_ref, w_ref, o_ref):
    o_ref[...] = _dot(x_ref[...], w_ref[...])


def _combine_pre_body(p_ref, h_ref, wroot_ref, b_ref, o_ref):
    agg = p_ref[pl.ds(0, N), :] + p_ref[pl.ds(NPAD, N), :]
    o_ref[...] = jnp.maximum(agg + b_ref[...] + _dot(h_ref[...], wroot_ref[...]), 0.0)


def _combine_post_body(p_ref, h_ref, wrel_ref, wroot_ref, b_ref, o_ref):
    agg = p_ref[pl.ds(0, N), :] + p_ref[pl.ds(NPAD, N), :]
    o_ref[...] = jnp.maximum(
        _dot(agg, wrel_ref[...]) + b_ref[...] + _dot(h_ref[...], wroot_ref[...]), 0.0)


def _combine_pre_mm_body(p_ref, h_ref, wroot_ref, b_ref, wnext_ref, o_ref, hw_ref):
    agg = p_ref[pl.ds(0, N), :] + p_ref[pl.ds(NPAD, N), :]
    h = jnp.maximum(agg + b_ref[...] + _dot(h_ref[...], wroot_ref[...]), 0.0)
    o_ref[...] = h
    hw_ref[...] = _dot(h, wnext_ref[...])


def _combine_post_mm_body(p_ref, h_ref, wrel_ref, wroot_ref, b_ref, wnext_ref,
                          o_ref, hw_ref):
    agg = p_ref[pl.ds(0, N), :] + p_ref[pl.ds(NPAD, N), :]
    h = jnp.maximum(
        _dot(agg, wrel_ref[...]) + b_ref[...] + _dot(h_ref[...], wroot_ref[...]), 0.0)
    o_ref[...] = h
    hw_ref[...] = _dot(h, wnext_ref[...])


def _head_body(p_ref, h_ref, wroot_ref, b_ref, batch_ref,
               w0, b0, w1, b1, w2, b2, w3, b3, o_ref):
    agg = p_ref[pl.ds(0, N), :] + p_ref[pl.ds(NPAD, N), :]
    h = jnp.maximum(agg + b_ref[...] + _dot(h_ref[...], wroot_ref[...]), 0.0)
    gid = lax.broadcasted_iota(jnp.int32, (NGRAPHS, N), 0)
    onehot = (gid == batch_ref[...]).astype(jnp.float32)
    sums = _dot(onehot, h)
    counts = jnp.sum(onehot, axis=1, keepdims=True)
    g = sums / jnp.maximum(counts, 1.0)
    g = jnp.maximum(_dot(g, w0[...]) + b0[...], 0.0)
    g = jnp.maximum(_dot(g, w1[...]) + b1[...], 0.0)
    g = jnp.maximum(_dot(g, w2[...]) + b2[...], 0.0)
    o_ref[...] = _dot(g, w3[...]) + b3[...]


def _tc(body, out_shapes):
    return pl.pallas_call(body, out_shape=out_shapes)


def kernel(x, edge_index, edge_attr, batch, w_rel0, b_rel0, w_root0, w_rel1, b_rel1, w_root1, w_rel2, b_rel2, w_root2, w_rel3, b_rel3, w_root3, w_rel4, b_rel4, w_root4, w_mlp0, b_mlp0, w_mlp1, b_mlp1, w_mlp2, b_mlp2, w_mlp3, b_mlp3):
    f32 = jnp.float32
    src = edge_index[0]
    dst = edge_index[1]
    pad = EPAD - E
    srcp = jnp.concatenate([src, jnp.zeros((pad,), jnp.int32)])
    attrp = jnp.concatenate([edge_attr, jnp.zeros((pad,), f32)])
    dstp = jnp.concatenate([dst, jnp.zeros((pad,), jnp.int32)]).reshape(EPAD // 128, 128)
    batch2d = batch.reshape(1, N)

    shp = lambda m, n: jax.ShapeDtypeStruct((m, n), f32)

    hw0 = _tc(_mm_body, shp(N, 32))(x, w_rel0.T)
    p0 = _make_edge_agg(32)(hw0, srcp, dstp, attrp)
    h1 = _tc(_combine_pre_body, shp(N, 32))(p0, x, w_root0.T, b_rel0.reshape(1, -1))

    p1 = _make_edge_agg(32)(h1, srcp, dstp, attrp)
    h2 = _tc(_combine_post_body, shp(N, 64))(
        p1, h1, w_rel1.T, w_root1.T, b_rel1.reshape(1, -1))

    p2 = _make_edge_agg(64)(h2, srcp, dstp, attrp)
    h3, hw3 = _tc(_combine_post_mm_body, (shp(N, 128), shp(N, 64)))(
        p2, h2, w_rel2.T, w_root2.T, b_rel2.reshape(1, -1), w_rel3.T)

    p3 = _make_edge_agg(64)(hw3, srcp, dstp, attrp)
    h4, hw4 = _tc(_combine_pre_mm_body, (shp(N, 64), shp(N, 32)))(
        p3, h3, w_root3.T, b_rel3.reshape(1, -1), w_rel4.T)

    p4 = _make_edge_agg(32)(hw4, srcp, dstp, attrp)

    out = _tc(_head_body, shp(NGRAPHS, 1))(
        p4, h4, w_root4.T, b_rel4.reshape(1, -1), batch2d,
        w_mlp0.T, b_mlp0.reshape(1, -1), w_mlp1.T, b_mlp1.reshape(1, -1),
        w_mlp2.T, b_mlp2.reshape(1, -1), w_mlp3.T, b_mlp3.reshape(1, -1))
    return out


# speedup vs baseline: 9.5325x; 1.2788x over previous
import functools

import jax
import jax.numpy as jnp
from jax import lax
from jax.experimental import pallas as pl
from jax.experimental.pallas import tpu as pltpu
from jax.experimental.pallas import tpu_sc as plsc

N = 10000
E = 320000
NGRAPHS = 64

NC = 2
NS = 16
NW = NC * NS
EPAD = 327680
EPW = EPAD // NW
NPAD = 10240
SUBN = NPAD // NS


_GDN = lax.GatherDimensionNumbers(offset_dims=(), collapsed_slice_dims=(0,),
                                  start_index_map=(0,))


def _bcast_lane(v, lane):
    idx = jnp.full((16, 1), lane, jnp.int32)
    return lax.gather(v, idx, _GDN, (1,),
                      mode=lax.GatherScatterMode.PROMISE_IN_BOUNDS)


def _edge_agg_body(d, B, stage_hw, hw_hbm, src_hbm, dst_hbm, attr_hbm, out_hbm,
                   hw_sh, acc_sh, src_v, dst_v, attr_v, rows0, rows1,
                   gs0, gs1, ss0, ss1):
    NCH = EPW // B
    cid = lax.axis_index("c")
    sid = lax.axis_index("s")
    wid = cid * NS + sid
    rows = (rows0, rows1)
    gsem = (gs0, gs1)
    ssem = (ss0, ss1)

    pltpu.sync_copy(src_hbm.at[pl.ds(wid * EPW, EPW)], src_v)
    pltpu.sync_copy(attr_hbm.at[pl.ds(wid * EPW, EPW)], attr_v)
    pltpu.sync_copy(dst_hbm.at[pl.ds(wid * (EPW // 128), EPW // 128)], dst_v)
    if stage_hw:
        pltpu.sync_copy(hw_hbm.at[pl.ds(sid * SUBN, SUBN)],
                        hw_sh.at[pl.ds(sid * SUBN, SUBN)])
    gsrc = hw_sh if stage_hw else hw_hbm

    nz = min(SUBN, B)

    @plsc.parallel_loop(0, nz)
    def zbody(i):
        for j in range(d // 16):
            rows0[i, pl.ds(j * 16, 16)] = jnp.zeros((16,), jnp.float32)
    off = 0
    while off < SUBN:
        cur = min(nz, SUBN - off)
        pltpu.sync_copy(rows0.at[pl.ds(0, cur)],
                        acc_sh.at[pl.ds(sid * SUBN + off, cur)])
        off += cur
    plsc.subcore_barrier()

    def fire_gather(c, k):
        pltpu.async_copy(gsrc.at[src_v.at[pl.ds(c * B, B)]], rows[k], gsem[k])

    def wait_gather(k):
        pltpu.make_async_copy(gsrc.at[src_v.at[pl.ds(0, B)]],
                              rows[k], gsem[k]).wait()

    def multiply(c, k):
        rk = rows[k]

        @plsc.parallel_loop(0, B // 16, unroll=2)
        def mbody(g):
            a16 = attr_v[pl.ds(c * B + g * 16, 16)]
            for e16 in range(16):
                a = _bcast_lane(a16, e16)
                e = g * 16 + e16
                for j in range(d // 16):
                    sl = pl.ds(j * 16, 16)
                    rk[e, sl] = rk[e, sl] * a

    def fire_scatters(c, k):
        for j in range(B // 128):
            pltpu.async_copy(rows[k].at[pl.ds(j * 128, 128)],
                             acc_sh.at[dst_v.at[c * (B // 128) + j]],
                             ssem[k], add=True)

    def drain_scatters(k):
        for j in range(B // 128):
            pltpu.make_async_copy(rows[k].at[pl.ds(j * 128, 128)],
                                  acc_sh.at[dst_v.at[0]], ssem[k]).wait()

    fire_gather(0, 0)
    fire_gather(1, 1)

    @pl.loop(0, NCH // 2)
    def round_loop(r):
        c0 = r * 2
        for k in (0, 1):
            wait_gather(k)
            multiply(c0 + k, k)
            fire_scatters(c0 + k, k)
        for k in (0, 1):
            drain_scatters(k)
            cn = jnp.minimum(c0 + k + 2, NCH - 1)
            fire_gather(cn, k)

    wait_gather(0)
    wait_gather(1)
    plsc.subcore_barrier()
    pltpu.sync_copy(acc_sh.at[pl.ds(sid * SUBN, SUBN)],
                    out_hbm.at[pl.ds(cid * NPAD + sid * SUBN, SUBN)])


@functools.cache
def _make_edge_agg(d):
    stage_hw = d <= 32
    B = 512 if d <= 32 else 256
    mesh = plsc.VectorSubcoreMesh(core_axis_name="c", subcore_axis_name="s",
                                  num_cores=NC, num_subcores=NS)
    return pl.kernel(
        functools.partial(_edge_agg_body, d, B, stage_hw),
        out_type=jax.ShapeDtypeStruct((NC * NPAD, d), jnp.float32),
        mesh=mesh,
        compiler_params=pltpu.CompilerParams(use_tc_tiling_on_sc=False),
        scratch_types=[
            pltpu.VMEM_SHARED((NPAD, d) if stage_hw else (8, d), jnp.float32),
            pltpu.VMEM_SHARED((NPAD, d), jnp.float32),
            pltpu.VMEM((EPW,), jnp.int32),
            pltpu.VMEM((EPW // 128, 128), jnp.int32),
            pltpu.VMEM((EPW,), jnp.float32),
            pltpu.VMEM((B, d), jnp.float32),
            pltpu.VMEM((B, d), jnp.float32),
            pltpu.SemaphoreType.DMA,
            pltpu.SemaphoreType.DMA,
            pltpu.SemaphoreType.DMA,
            pltpu.SemaphoreType.DMA,
        ],
    )


def _dot(a, b):
    return lax.dot_general(a, b, (((1,), (0,)), ((), ())),
                           preferred_element_type=jnp.float32)


def _mm_body(x_ref, w_ref, o_ref):
    o_ref[pl.ds(0, N), :] = _dot(x_ref[...], w_ref[...])


def _combine_pre_body(p_ref, h_ref, wroot_ref, b_ref, o_ref):
    agg = p_ref[pl.ds(0, N), :] + p_ref[pl.ds(NPAD, N), :]
    o_ref[pl.ds(0, N), :] = jnp.maximum(
        agg + b_ref[...] + _dot(h_ref[pl.ds(0, N), :], wroot_ref[...]), 0.0)


def _combine_post_body(p_ref, h_ref, wrel_ref, wroot_ref, b_ref, o_ref):
    agg = p_ref[pl.ds(0, N), :] + p_ref[pl.ds(NPAD, N), :]
    o_ref[pl.ds(0, N), :] = jnp.maximum(
        _dot(agg, wrel_ref[...]) + b_ref[...]
        + _dot(h_ref[pl.ds(0, N), :], wroot_ref[...]), 0.0)


def _combine_post_mm_body(p_ref, h_ref, wrel_ref, wroot_ref, b_ref, wnext_ref,
                          o_ref, hw_ref):
    agg = p_ref[pl.ds(0, N), :] + p_ref[pl.ds(NPAD, N), :]
    h = jnp.maximum(
        _dot(agg, wrel_ref[...]) + b_ref[...]
        + _dot(h_ref[pl.ds(0, N), :], wroot_ref[...]), 0.0)
    o_ref[pl.ds(0, N), :] = h
    hw_ref[pl.ds(0, N), :] = _dot(h, wnext_ref[...])


def _combine_pre_mm_body(p_ref, h_ref, wroot_ref, b_ref, wnext_ref,
                         o_ref, hw_ref):
    agg = p_ref[pl.ds(0, N), :] + p_ref[pl.ds(NPAD, N), :]
    h = jnp.maximum(
        agg + b_ref[...] + _dot(h_ref[pl.ds(0, N), :], wroot_ref[...]), 0.0)
    o_ref[pl.ds(0, N), :] = h
    hw_ref[pl.ds(0, N), :] = _dot(h, wnext_ref[...])


def _head_body(p_ref, h_ref, wroot_ref, b_ref, batch_ref,
               w0, b0, w1, b1, w2, b2, w3, b3, o_ref):
    agg = p_ref[pl.ds(0, N), :] + p_ref[pl.ds(NPAD, N), :]
    h = jnp.maximum(
        agg + b_ref[...] + _dot(h_ref[pl.ds(0, N), :], wroot_ref[...]), 0.0)
    gid = lax.broadcasted_iota(jnp.int32, (NGRAPHS, N), 0)
    onehot = (gid == batch_ref[...]).astype(jnp.float32)
    sums = _dot(onehot, h)
    counts = jnp.sum(onehot, axis=1, keepdims=True)
    g = sums / jnp.maximum(counts, 1.0)
    g = jnp.maximum(_dot(g, w0[...]) + b0[...], 0.0)
    g = jnp.maximum(_dot(g, w1[...]) + b1[...], 0.0)
    g = jnp.maximum(_dot(g, w2[...]) + b2[...], 0.0)
    o_ref[...] = _dot(g, w3[...]) + b3[...]


def _tc(body, out_shapes):
    return pl.pallas_call(body, out_shape=out_shapes)


def kernel(x, edge_index, edge_attr, batch, w_rel0, b_rel0, w_root0, w_rel1, b_rel1, w_root1, w_rel2, b_rel2, w_root2, w_rel3, b_rel3, w_root3, w_rel4, b_rel4, w_root4, w_mlp0, b_mlp0, w_mlp1, b_mlp1, w_mlp2, b_mlp2, w_mlp3, b_mlp3):
    f32 = jnp.float32
    src = edge_index[0]
    dst = edge_index[1]
    pad = EPAD - E
    srcp = jnp.concatenate([src, jnp.zeros((pad,), jnp.int32)])
    attrp = jnp.concatenate([edge_attr, jnp.zeros((pad,), f32)])
    dstp = jnp.concatenate([dst, jnp.zeros((pad,), jnp.int32)]).reshape(EPAD // 128, 128)
    batch2d = batch.reshape(1, N)

    shp = lambda m, n: jax.ShapeDtypeStruct((m, n), f32)

    hw0 = _tc(_mm_body, shp(NPAD, 32))(x, w_rel0.T)
    p0 = _make_edge_agg(32)(hw0, srcp, dstp, attrp)
    h1 = _tc(_combine_pre_body, shp(NPAD, 32))(p0, x, w_root0.T, b_rel0.reshape(1, -1))

    p1 = _make_edge_agg(32)(h1, srcp, dstp, attrp)
    h2 = _tc(_combine_post_body, shp(NPAD, 64))(
        p1, h1, w_rel1.T, w_root1.T, b_rel1.reshape(1, -1))

    p2 = _make_edge_agg(64)(h2, srcp, dstp, attrp)
    h3, hw3 = _tc(_combine_post_mm_body, (shp(NPAD, 128), shp(NPAD, 64)))(
        p2, h2, w_rel2.T, w_root2.T, b_rel2.reshape(1, -1), w_rel3.T)

    p3 = _make_edge_agg(64)(hw3, srcp, dstp, attrp)
    h4, hw4 = _tc(_combine_pre_mm_body, (shp(NPAD, 64), shp(NPAD, 32)))(
        p3, h3, w_root3.T, b_rel3.reshape(1, -1), w_rel4.T)

    p4 = _make_edge_agg(32)(hw4, srcp, dstp, attrp)

    out = _tc(_head_body, shp(NGRAPHS, 1))(
        p4, h4, w_root4.T, b_rel4.reshape(1, -1), batch2d,
        w_mlp0.T, b_mlp0.reshape(1, -1), w_mlp1.T, b_mlp1.reshape(1, -1),
        w_mlp2.T, b_mlp2.reshape(1, -1), w_mlp3.T, b_mlp3.reshape(1, -1))
    return out


# speedup vs baseline: 15.4098x; 1.6165x over previous
import functools

import jax
import jax.numpy as jnp
from jax import lax
from jax.experimental import pallas as pl
from jax.experimental.pallas import tpu as pltpu
from jax.experimental.pallas import tpu_sc as plsc

N = 10000
E = 320000
NGRAPHS = 64

NC = 2
NS = 16
NW = NC * NS
EPAD = 327680
EPW = EPAD // NW
NPAD = 10240
SUBN = NPAD // NS


_GDN = lax.GatherDimensionNumbers(offset_dims=(), collapsed_slice_dims=(0,),
                                  start_index_map=(0,))


def _bcast_lane(v, lane):
    idx = jnp.full((16, 1), lane, jnp.int32)
    return lax.gather(v, idx, _GDN, (1,),
                      mode=lax.GatherScatterMode.PROMISE_IN_BOUNDS)


def _edge_agg_body(d, B, stage_hw, hw_hbm, src_hbm, dst_hbm, attr_hbm, out_hbm,
                   hw_sh, acc_sh, src_v, dst_v, attr_v, rows0, rows1,
                   gs0, gs1, ss0, ss1):
    NCH = EPW // B
    cid = lax.axis_index("c")
    sid = lax.axis_index("s")
    wid = cid * NS + sid
    rows = (rows0, rows1)
    gsem = (gs0, gs1)
    ssem = (ss0, ss1)

    pltpu.sync_copy(src_hbm.at[pl.ds(wid * EPW, EPW)], src_v)
    pltpu.sync_copy(attr_hbm.at[pl.ds(wid * EPW, EPW)], attr_v)
    pltpu.sync_copy(dst_hbm.at[pl.ds(wid * (EPW // 128), EPW // 128)], dst_v)
    if stage_hw:
        pltpu.sync_copy(hw_hbm.at[pl.ds(sid * SUBN, SUBN)],
                        hw_sh.at[pl.ds(sid * SUBN, SUBN)])
    gsrc = hw_sh if stage_hw else hw_hbm

    nz = min(SUBN, B)

    @plsc.parallel_loop(0, nz)
    def zbody(i):
        for j in range(d // 16):
            rows0[i, pl.ds(j * 16, 16)] = jnp.zeros((16,), jnp.float32)
    off = 0
    while off < SUBN:
        cur = min(nz, SUBN - off)
        pltpu.sync_copy(rows0.at[pl.ds(0, cur)],
                        acc_sh.at[pl.ds(sid * SUBN + off, cur)])
        off += cur
    plsc.subcore_barrier()

    def fire_gather(c, k):
        pltpu.async_copy(gsrc.at[src_v.at[pl.ds(c * B, B)]], rows[k], gsem[k])

    def wait_gather(k):
        pltpu.make_async_copy(gsrc.at[src_v.at[pl.ds(0, B)]],
                              rows[k], gsem[k]).wait()

    def multiply(c, k):
        rk = rows[k]

        @plsc.parallel_loop(0, B // 16, unroll=2)
        def mbody(g):
            a16 = attr_v[pl.ds(c * B + g * 16, 16)]
            for e16 in range(16):
                a = _bcast_lane(a16, e16)
                e = g * 16 + e16
                for j in range(d // 16):
                    sl = pl.ds(j * 16, 16)
                    rk[e, sl] = rk[e, sl] * a

    def fire_scatters(c, k):
        for j in range(B // 128):
            pltpu.async_copy(rows[k].at[pl.ds(j * 128, 128)],
                             acc_sh.at[dst_v.at[c * (B // 128) + j]],
                             ssem[k], add=True)

    def drain_scatters(k):
        for j in range(B // 128):
            pltpu.make_async_copy(rows[k].at[pl.ds(j * 128, 128)],
                                  acc_sh.at[dst_v.at[0]], ssem[k]).wait()

    fire_gather(0, 0)
    fire_gather(1, 1)

    @pl.loop(0, NCH // 2)
    def round_loop(r):
        c0 = r * 2
        for k in (0, 1):
            wait_gather(k)
            multiply(c0 + k, k)
            fire_scatters(c0 + k, k)
        for k in (0, 1):
            drain_scatters(k)
            cn = jnp.minimum(c0 + k + 2, NCH - 1)
            fire_gather(cn, k)

    wait_gather(0)
    wait_gather(1)
    plsc.subcore_barrier()
    pltpu.sync_copy(acc_sh.at[pl.ds(sid * SUBN, SUBN)],
                    out_hbm.at[pl.ds(cid * NPAD + sid * SUBN, SUBN)])


@functools.cache
def _make_edge_agg(d):
    stage_hw = True
    B = 512 if d <= 32 else 128
    mesh = plsc.VectorSubcoreMesh(core_axis_name="c", subcore_axis_name="s",
                                  num_cores=NC, num_subcores=NS)
    return pl.kernel(
        functools.partial(_edge_agg_body, d, B, stage_hw),
        out_type=jax.ShapeDtypeStruct((NC * NPAD, d), jnp.float32),
        mesh=mesh,
        compiler_params=pltpu.CompilerParams(use_tc_tiling_on_sc=False),
        scratch_types=[
            pltpu.VMEM_SHARED((NPAD, d) if stage_hw else (8, d), jnp.float32),
            pltpu.VMEM_SHARED((NPAD, d), jnp.float32),
            pltpu.VMEM((EPW,), jnp.int32),
            pltpu.VMEM((EPW // 128, 128), jnp.int32),
            pltpu.VMEM((EPW,), jnp.float32),
            pltpu.VMEM((B, d), jnp.float32),
            pltpu.VMEM((B, d), jnp.float32),
            pltpu.SemaphoreType.DMA,
            pltpu.SemaphoreType.DMA,
            pltpu.SemaphoreType.DMA,
            pltpu.SemaphoreType.DMA,
        ],
    )


def _dot(a, b):
    return lax.dot_general(a, b, (((1,), (0,)), ((), ())),
                           preferred_element_type=jnp.float32)


def _mm_body(x_ref, w_ref, o_ref):
    o_ref[pl.ds(0, N), :] = _dot(x_ref[...], w_ref[...])


def _combine_pre_body(p_ref, h_ref, wroot_ref, b_ref, o_ref):
    agg = p_ref[pl.ds(0, N), :] + p_ref[pl.ds(NPAD, N), :]
    o_ref[pl.ds(0, N), :] = jnp.maximum(
        agg + b_ref[...] + _dot(h_ref[pl.ds(0, N), :], wroot_ref[...]), 0.0)


def _combine_post_body(p_ref, h_ref, wrel_ref, wroot_ref, b_ref, o_ref):
    agg = p_ref[pl.ds(0, N), :] + p_ref[pl.ds(NPAD, N), :]
    o_ref[pl.ds(0, N), :] = jnp.maximum(
        _dot(agg, wrel_ref[...]) + b_ref[...]
        + _dot(h_ref[pl.ds(0, N), :], wroot_ref[...]), 0.0)


def _combine_post_mm_body(p_ref, h_ref, wrel_ref, wroot_ref, b_ref, wnext_ref,
                          o_ref, hw_ref):
    agg = p_ref[pl.ds(0, N), :] + p_ref[pl.ds(NPAD, N), :]
    h = jnp.maximum(
        _dot(agg, wrel_ref[...]) + b_ref[...]
        + _dot(h_ref[pl.ds(0, N), :], wroot_ref[...]), 0.0)
    o_ref[pl.ds(0, N), :] = h
    hw_ref[pl.ds(0, N), :] = _dot(h, wnext_ref[...])


def _combine_pre_mm_body(p_ref, h_ref, wroot_ref, b_ref, wnext_ref,
                         o_ref, hw_ref):
    agg = p_ref[pl.ds(0, N), :] + p_ref[pl.ds(NPAD, N), :]
    h = jnp.maximum(
        agg + b_ref[...] + _dot(h_ref[pl.ds(0, N), :], wroot_ref[...]), 0.0)
    o_ref[pl.ds(0, N), :] = h
    hw_ref[pl.ds(0, N), :] = _dot(h, wnext_ref[...])


def _head_body(p_ref, h_ref, wroot_ref, b_ref, batch_ref,
               w0, b0, w1, b1, w2, b2, w3, b3, o_ref):
    agg = p_ref[pl.ds(0, N), :] + p_ref[pl.ds(NPAD, N), :]
    h = jnp.maximum(
        agg + b_ref[...] + _dot(h_ref[pl.ds(0, N), :], wroot_ref[...]), 0.0)
    gid = lax.broadcasted_iota(jnp.int32, (NGRAPHS, N), 0)
    onehot = (gid == batch_ref[...]).astype(jnp.float32)
    sums = _dot(onehot, h)
    counts = jnp.sum(onehot, axis=1, keepdims=True)
    g = sums / jnp.maximum(counts, 1.0)
    g = jnp.maximum(_dot(g, w0[...]) + b0[...], 0.0)
    g = jnp.maximum(_dot(g, w1[...]) + b1[...], 0.0)
    g = jnp.maximum(_dot(g, w2[...]) + b2[...], 0.0)
    o_ref[...] = _dot(g, w3[...]) + b3[...]


def _tc(body, out_shapes):
    return pl.pallas_call(body, out_shape=out_shapes)


def kernel(x, edge_index, edge_attr, batch, w_rel0, b_rel0, w_root0, w_rel1, b_rel1, w_root1, w_rel2, b_rel2, w_root2, w_rel3, b_rel3, w_root3, w_rel4, b_rel4, w_root4, w_mlp0, b_mlp0, w_mlp1, b_mlp1, w_mlp2, b_mlp2, w_mlp3, b_mlp3):
    f32 = jnp.float32
    src = edge_index[0]
    dst = edge_index[1]
    pad = EPAD - E
    srcp = jnp.concatenate([src, jnp.zeros((pad,), jnp.int32)])
    attrp = jnp.concatenate([edge_attr, jnp.zeros((pad,), f32)])
    dstp = jnp.concatenate([dst, jnp.zeros((pad,), jnp.int32)]).reshape(EPAD // 128, 128)
    batch2d = batch.reshape(1, N)

    shp = lambda m, n: jax.ShapeDtypeStruct((m, n), f32)

    hw0 = _tc(_mm_body, shp(NPAD, 32))(x, w_rel0.T)
    p0 = _make_edge_agg(32)(hw0, srcp, dstp, attrp)
    h1 = _tc(_combine_pre_body, shp(NPAD, 32))(p0, x, w_root0.T, b_rel0.reshape(1, -1))

    p1 = _make_edge_agg(32)(h1, srcp, dstp, attrp)
    h2 = _tc(_combine_post_body, shp(NPAD, 64))(
        p1, h1, w_rel1.T, w_root1.T, b_rel1.reshape(1, -1))

    p2 = _make_edge_agg(64)(h2, srcp, dstp, attrp)
    h3, hw3 = _tc(_combine_post_mm_body, (shp(NPAD, 128), shp(NPAD, 64)))(
        p2, h2, w_rel2.T, w_root2.T, b_rel2.reshape(1, -1), w_rel3.T)

    p3 = _make_edge_agg(64)(hw3, srcp, dstp, attrp)
    h4, hw4 = _tc(_combine_pre_mm_body, (shp(NPAD, 64), shp(NPAD, 32)))(
        p3, h3, w_root3.T, b_rel3.reshape(1, -1), w_rel4.T)

    p4 = _make_edge_agg(32)(hw4, srcp, dstp, attrp)

    out = _tc(_head_body, shp(NGRAPHS, 1))(
        p4, h4, w_root4.T, b_rel4.reshape(1, -1), batch2d,
        w_mlp0.T, b_mlp0.reshape(1, -1), w_mlp1.T, b_mlp1.reshape(1, -1),
        w_mlp2.T, b_mlp2.reshape(1, -1), w_mlp3.T, b_mlp3.reshape(1, -1))
    return out


# speedup vs baseline: 16.9379x; 1.0992x over previous
import functools

import jax
import jax.numpy as jnp
import numpy as np
from jax import lax
from jax.experimental import pallas as pl
from jax.experimental.pallas import tpu as pltpu
from jax.experimental.pallas import tpu_sc as plsc

N = 10000
E = 320000
NGRAPHS = 64

NC = 2
NS = 16
NW = NC * NS
EPAD = 327680
EPW = EPAD // NW
NPAD = 10240
SUBN = NPAD // NS


def _perm(d):
    p = np.zeros((d,), np.int64)
    for g in range(d // 32):
        for i in range(16):
            p[g * 32 + 2 * i] = g * 32 + i
            p[g * 32 + 2 * i + 1] = g * 32 + 16 + i
    return p


def _pmat(d):
    return jnp.asarray(np.eye(d, dtype=np.float32)[_perm(d)].T)


_GDN = lax.GatherDimensionNumbers(offset_dims=(), collapsed_slice_dims=(0,),
                                  start_index_map=(0,))


def _bcast_lane(v, lane):
    idx = jnp.full((16, 1), lane, jnp.int32)
    return lax.gather(v, idx, _GDN, (1,),
                      mode=lax.GatherScatterMode.PROMISE_IN_BOUNDS)


def _edge_agg_body(d, B, hw_hbm, src_hbm, dst_hbm, attr_hbm, out_hbm,
                   hw_sh, acc_sh, src_v, dst_v, attr_v,
                   braw0, braw1, rsc0, rsc1, gs0, gs1, ss0, ss1):
    NCH = EPW // B
    cid = lax.axis_index("c")
    sid = lax.axis_index("s")
    wid = cid * NS + sid
    braw = (braw0, braw1)
    rsc = (rsc0, rsc1)
    gsem = (gs0, gs1)
    ssem = (ss0, ss1)

    pltpu.sync_copy(src_hbm.at[pl.ds(wid * EPW, EPW)], src_v)
    pltpu.sync_copy(attr_hbm.at[pl.ds(wid * EPW, EPW)], attr_v)
    pltpu.sync_copy(dst_hbm.at[pl.ds(wid * (EPW // 128), EPW // 128)], dst_v)
    pltpu.sync_copy(hw_hbm.at[pl.ds(sid * SUBN, SUBN)],
                    hw_sh.at[pl.ds(sid * SUBN, SUBN)])

    nz = min(SUBN, B)

    @plsc.parallel_loop(0, nz)
    def zbody(i):
        for j in range(d // 16):
            rsc0[i, pl.ds(j * 16, 16)] = jnp.zeros((16,), jnp.float32)
    off = 0
    while off < SUBN:
        cur = min(nz, SUBN - off)
        pltpu.sync_copy(rsc0.at[pl.ds(0, cur)],
                        acc_sh.at[pl.ds(sid * SUBN + off, cur)])
        off += cur
    plsc.subcore_barrier()

    def fire_gather(c, k):
        pltpu.async_copy(hw_sh.at[src_v.at[pl.ds(c * B, B)]], braw[k], gsem[k])

    def wait_gather(k):
        pltpu.make_async_copy(hw_sh.at[src_v.at[pl.ds(0, B)]],
                              braw[k], gsem[k]).wait()

    def multiply(c, k):
        rb = braw[k]
        ro = rsc[k]

        @plsc.parallel_loop(0, B // 16, unroll=2)
        def mbody(g):
            a16 = attr_v[pl.ds(c * B + g * 16, 16)]
            for e16 in range(16):
                a = _bcast_lane(a16, e16)
                e = g * 16 + e16
                for j in range(d // 32):
                    bf = rb[e, pl.ds(j * 32, 32)]
                    lo, hi = plsc.unpack(bf, format=plsc.PackFormat.INTERLEAVED)
                    ro[e, pl.ds(j * 32, 16)] = lo * a
                    ro[e, pl.ds(j * 32 + 16, 16)] = hi * a

    def fire_scatters(c, k):
        for j in range(B // 128):
            pltpu.async_copy(rsc[k].at[pl.ds(j * 128, 128)],
                             acc_sh.at[dst_v.at[c * (B // 128) + j]],
                             ssem[k], add=True)

    def drain_scatters(k):
        for j in range(B // 128):
            pltpu.make_async_copy(rsc[k].at[pl.ds(j * 128, 128)],
                                  acc_sh.at[dst_v.at[0]], ssem[k]).wait()

    fire_gather(0, 0)
    fire_gather(1, 1)

    @pl.loop(0, NCH // 2)
    def round_loop(r):
        c0 = r * 2
        for k in (0, 1):
            wait_gather(k)
            multiply(c0 + k, k)
            fire_scatters(c0 + k, k)
        for k in (0, 1):
            drain_scatters(k)
            cn = jnp.minimum(c0 + k + 2, NCH - 1)
            fire_gather(cn, k)

    wait_gather(0)
    wait_gather(1)
    plsc.subcore_barrier()
    pltpu.sync_copy(acc_sh.at[pl.ds(sid * SUBN, SUBN)],
                    out_hbm.at[pl.ds(cid * NPAD + sid * SUBN, SUBN)])


@functools.cache
def _make_edge_agg(d):
    B = 512 if d <= 32 else 128
    mesh = plsc.VectorSubcoreMesh(core_axis_name="c", subcore_axis_name="s",
                                  num_cores=NC, num_subcores=NS)
    return pl.kernel(
        functools.partial(_edge_agg_body, d, B),
        out_type=jax.ShapeDtypeStruct((NC * NPAD, d), jnp.float32),
        mesh=mesh,
        compiler_params=pltpu.CompilerParams(use_tc_tiling_on_sc=False,
                                             needs_layout_passes=False),
        scratch_types=[
            pltpu.VMEM_SHARED((NPAD, d), jnp.bfloat16),
            pltpu.VMEM_SHARED((NPAD, d), jnp.float32),
            pltpu.VMEM((EPW,), jnp.int32),
            pltpu.VMEM((EPW // 128, 128), jnp.int32),
            pltpu.VMEM((EPW,), jnp.float32),
            pltpu.VMEM((B, d), jnp.bfloat16),
            pltpu.VMEM((B, d), jnp.bfloat16),
            pltpu.VMEM((B, d), jnp.float32),
            pltpu.VMEM((B, d), jnp.float32),
            pltpu.SemaphoreType.DMA,
            pltpu.SemaphoreType.DMA,
            pltpu.SemaphoreType.DMA,
            pltpu.SemaphoreType.DMA,
        ],
    )


def _dot(a, b):
    return lax.dot_general(a, b, (((1,), (0,)), ((), ())),
                           preferred_element_type=jnp.float32)


def _mm_body(x_ref, w_ref, o_ref):
    o_ref[pl.ds(0, N), :] = _dot(x_ref[...], w_ref[...]).astype(jnp.bfloat16)


def _combine_pre_body(p_ref, h_ref, wroot_ref, b_ref, pmat_ref, o_ref, t_ref):
    agg = p_ref[pl.ds(0, N), :] + p_ref[pl.ds(NPAD, N), :]
    h = jnp.maximum(
        agg + b_ref[...] + _dot(h_ref[pl.ds(0, N), :], wroot_ref[...]), 0.0)
    o_ref[pl.ds(0, N), :] = h
    t_ref[pl.ds(0, N), :] = _dot(h, pmat_ref[...]).astype(jnp.bfloat16)


def _combine_post_body(p_ref, h_ref, wrel_ref, wroot_ref, b_ref, pmat_ref,
                       o_ref, t_ref):
    agg = p_ref[pl.ds(0, N), :] + p_ref[pl.ds(NPAD, N), :]
    h = jnp.maximum(
        _dot(agg, wrel_ref[...]) + b_ref[...]
        + _dot(h_ref[pl.ds(0, N), :], wroot_ref[...]), 0.0)
    o_ref[pl.ds(0, N), :] = h
    t_ref[pl.ds(0, N), :] = _dot(h, pmat_ref[...]).astype(jnp.bfloat16)


def _combine_post_mm_body(p_ref, h_ref, wrel_ref, wroot_ref, b_ref, wnext_ref,
                          o_ref, t_ref):
    agg = p_ref[pl.ds(0, N), :] + p_ref[pl.ds(NPAD, N), :]
    h = jnp.maximum(
        _dot(agg, wrel_ref[...]) + b_ref[...]
        + _dot(h_ref[pl.ds(0, N), :], wroot_ref[...]), 0.0)
    o_ref[pl.ds(0, N), :] = h
    t_ref[pl.ds(0, N), :] = _dot(h, wnext_ref[...]).astype(jnp.bfloat16)


def _combine_pre_mm_body(p_ref, h_ref, wroot_ref, b_ref, wnext_ref,
                         o_ref, t_ref):
    agg = p_ref[pl.ds(0, N), :] + p_ref[pl.ds(NPAD, N), :]
    h = jnp.maximum(
        agg + b_ref[...] + _dot(h_ref[pl.ds(0, N), :], wroot_ref[...]), 0.0)
    o_ref[pl.ds(0, N), :] = h
    t_ref[pl.ds(0, N), :] = _dot(h, wnext_ref[...]).astype(jnp.bfloat16)


def _head_body(p_ref, h_ref, wroot_ref, b_ref, batch_ref,
               w0, b0, w1, b1, w2, b2, w3, b3, o_ref):
    agg = p_ref[pl.ds(0, N), :] + p_ref[pl.ds(NPAD, N), :]
    h = jnp.maximum(
        agg + b_ref[...] + _dot(h_ref[pl.ds(0, N), :], wroot_ref[...]), 0.0)
    gid = lax.broadcasted_iota(jnp.int32, (NGRAPHS, N), 0)
    onehot = (gid == batch_ref[...]).astype(jnp.float32)
    sums = _dot(onehot, h)
    counts = jnp.sum(onehot, axis=1, keepdims=True)
    g = sums / jnp.maximum(counts, 1.0)
    g = jnp.maximum(_dot(g, w0[...]) + b0[...], 0.0)
    g = jnp.maximum(_dot(g, w1[...]) + b1[...], 0.0)
    g = jnp.maximum(_dot(g, w2[...]) + b2[...], 0.0)
    o_ref[...] = _dot(g, w3[...]) + b3[...]


def _tc(body, out_shapes):
    return pl.pallas_call(body, out_shape=out_shapes)


def kernel(x, edge_index, edge_attr, batch, w_rel0, b_rel0, w_root0, w_rel1, b_rel1, w_root1, w_rel2, b_rel2, w_root2, w_rel3, b_rel3, w_root3, w_rel4, b_rel4, w_root4, w_mlp0, b_mlp0, w_mlp1, b_mlp1, w_mlp2, b_mlp2, w_mlp3, b_mlp3):
    f32 = jnp.float32
    src = edge_index[0]
    dst = edge_index[1]
    pad = EPAD - E
    srcp = jnp.concatenate([src, jnp.zeros((pad,), jnp.int32)])
    attrp = jnp.concatenate([edge_attr, jnp.zeros((pad,), f32)])
    dstp = jnp.concatenate([dst, jnp.zeros((pad,), jnp.int32)]).reshape(EPAD // 128, 128)
    batch2d = batch.reshape(1, N)
    p32 = _perm(32)
    p64 = _perm(64)

    shp = lambda m, n: jax.ShapeDtypeStruct((m, n), f32)
    bshp = lambda m, n: jax.ShapeDtypeStruct((m, n), jnp.bfloat16)

    t0 = _tc(_mm_body, bshp(NPAD, 32))(x, w_rel0.T[:, p32])
    q0 = _make_edge_agg(32)(t0, srcp, dstp, attrp)
    h1, t1 = _tc(_combine_pre_body, (shp(NPAD, 32), bshp(NPAD, 32)))(
        q0, x, w_root0.T, b_rel0.reshape(1, -1), _pmat(32))

    q1 = _make_edge_agg(32)(t1, srcp, dstp, attrp)
    h2, t2 = _tc(_combine_post_body, (shp(NPAD, 64), bshp(NPAD, 64)))(
        q1, h1, w_rel1.T, w_root1.T, b_rel1.reshape(1, -1), _pmat(64))

    q2 = _make_edge_agg(64)(t2, srcp, dstp, attrp)
    h3, t3 = _tc(_combine_post_mm_body, (shp(NPAD, 128), bshp(NPAD, 64)))(
        q2, h2, w_rel2.T, w_root2.T, b_rel2.reshape(1, -1), w_rel3.T[:, p64])

    q3 = _make_edge_agg(64)(t3, srcp, dstp, attrp)
    h4, t4 = _tc(_combine_pre_mm_body, (shp(NPAD, 64), bshp(NPAD, 32)))(
        q3, h3, w_root3.T, b_rel3.reshape(1, -1), w_rel4.T[:, p32])

    q4 = _make_edge_agg(32)(t4, srcp, dstp, attrp)

    out = _tc(_head_body, shp(NGRAPHS, 1))(
        q4, h4, w_root4.T, b_rel4.reshape(1, -1), batch2d,
        w_mlp0.T, b_mlp0.reshape(1, -1), w_mlp1.T, b_mlp1.reshape(1, -1),
        w_mlp2.T, b_mlp2.reshape(1, -1), w_mlp3.T, b_mlp3.reshape(1, -1))
    return out
